# Initial kernel scaffold; baseline (speedup 1.0000x reference)
import jax
import jax.numpy as jnp
from jax.experimental import pallas as pl


def kernel(x, x_pad, idx, neighbor_pts, query_pts, W, b, gamma, beta):
    raise NotImplementedError("write your pallas kernel here")


# speedup vs baseline: 3.1531x; 3.1531x over previous
import functools

import jax
import jax.numpy as jnp
from jax import lax
from jax.experimental import pallas as pl
from jax.experimental.pallas import tpu as pltpu
from jax.experimental.pallas import tpu_sc as plsc

N = 10000
K = 32
C = 128
R = 0.1
NV = C // 16

N_PAD = 10240
NC = 2
NS = 16
NW = NC * NS
PW = N_PAD // NW
SS = PW // 16
G = 4
NGRP = PW // G

_SIGMA = R * 0.3
_WSCALE = -1.0 / (2.0 * _SIGMA * _SIGMA + 1e-9)

_BLK = 1280


def _dense_body(x_ref, w_ref, b_ref, g_ref, bt_ref, xpad_ref, o_ref):
    i = pl.program_id(0)
    h = lax.dot_general(x_ref[...], w_ref[...], (((1,), (1,)), ((), ())),
                        preferred_element_type=jnp.float32)
    h = h + b_ref[...]
    mu = jnp.mean(h, axis=-1, keepdims=True)
    var = jnp.mean((h - mu) ** 2, axis=-1, keepdims=True)
    h = (h - mu) * lax.rsqrt(var + 1e-5) * g_ref[...] + bt_ref[...]
    h = jnp.where(h >= 0, h, 0.1 * h)
    rows = i * _BLK + lax.broadcasted_iota(jnp.int32, (_BLK, 1), 0)
    o_ref[...] = jnp.where(rows >= N, xpad_ref[...], h)


def _dense(x_padded, W, b, gamma, beta, x_pad):
    return pl.pallas_call(
        _dense_body,
        grid=(N_PAD // _BLK,),
        in_specs=[
            pl.BlockSpec((_BLK, C), lambda i: (i, 0)),
            pl.BlockSpec((C, C), lambda i: (0, 0)),
            pl.BlockSpec((1, C), lambda i: (0, 0)),
            pl.BlockSpec((1, C), lambda i: (0, 0)),
            pl.BlockSpec((1, C), lambda i: (0, 0)),
            pl.BlockSpec((1, C), lambda i: (0, 0)),
        ],
        out_specs=pl.BlockSpec((_BLK, C), lambda i: (i, 0)),
        out_shape=jax.ShapeDtypeStruct((N_PAD, C), jnp.float32),
    )(x_padded, W, b, gamma, beta, x_pad)


_mesh = plsc.VectorSubcoreMesh(core_axis_name="c", subcore_axis_name="s")


@functools.partial(
    pl.kernel,
    out_type=jax.ShapeDtypeStruct((N_PAD, C), jnp.float32),
    mesh=_mesh,
    scratch_types=[
        pltpu.VMEM((N_PAD,), jnp.float32),
        pltpu.VMEM((N_PAD,), jnp.float32),
        pltpu.VMEM((N_PAD,), jnp.float32),
        pltpu.VMEM((PW * K,), jnp.int32),
        pltpu.VMEM((PW * K,), jnp.int32),
        pltpu.VMEM((PW,), jnp.float32),
        pltpu.VMEM((PW,), jnp.float32),
        pltpu.VMEM((PW,), jnp.float32),
        pltpu.VMEM((PW * K,), jnp.float32),
        pltpu.VMEM((2, G * K, C), jnp.float32),
        pltpu.VMEM((G, C), jnp.float32),
        pltpu.SemaphoreType.DMA,
        pltpu.SemaphoreType.DMA,
    ],
    compiler_params=pltpu.CompilerParams(needs_layout_passes=False),
)
def _sc_pool(table_hbm, idx_hbm, idxt_hbm, px_hbm, py_hbm, pz_hbm,
             qx_hbm, qy_hbm, qz_hbm, out_hbm,
             px_v, py_v, pz_v, idx_v, idxt_v, qx_v, qy_v, qz_v,
             w_v, rows_v, out_v, sem0, sem1):
    wid = lax.axis_index("s") * NC + lax.axis_index("c")
    pbase = wid * PW

    pltpu.sync_copy(px_hbm, px_v)
    pltpu.sync_copy(py_hbm, py_v)
    pltpu.sync_copy(pz_hbm, pz_v)
    pltpu.sync_copy(idx_hbm.at[pl.ds(pbase * K, PW * K)], idx_v)
    pltpu.sync_copy(idxt_hbm.at[pl.ds(pbase * K, PW * K)], idxt_v)
    pltpu.sync_copy(qx_hbm.at[pl.ds(pbase, PW)], qx_v)
    pltpu.sync_copy(qy_hbm.at[pl.ds(pbase, PW)], qy_v)
    pltpu.sync_copy(qz_hbm.at[pl.ds(pbase, PW)], qz_v)

    def _start(g, buf, sem):
        idx_slice = idx_v.at[pl.ds(g * (G * K), G * K)]
        pltpu.async_copy(table_hbm.at[idx_slice], rows_v.at[buf], sem)

    def _wait(g, buf, sem):
        pltpu.make_async_copy(
            table_hbm.at[idx_v.at[pl.ds(g * (G * K), G * K)]],
            rows_v.at[buf], sem).wait()

    _start(0, 0, sem0)
    _start(1, 1, sem1)

    lanes_k = lax.iota(jnp.int32, 16) * K

    def _weights_body(s, carry):
        soff = pl.multiple_of(s * 16, 16)
        qx = qx_v[pl.ds(soff, 16)]
        qy = qy_v[pl.ds(soff, 16)]
        qz = qz_v[pl.ds(soff, 16)]
        den = jnp.zeros((16,), jnp.float32)
        wbase = s * (16 * K)
        for j in range(K):
            ivec = idxt_v[pl.ds(pl.multiple_of(j * PW + s * 16, 16), 16)]
            dx = qx - plsc.load_gather(px_v, [ivec])
            dy = qy - plsc.load_gather(py_v, [ivec])
            dz = qz - plsc.load_gather(pz_v, [ivec])
            d = dx * dx + dy * dy + dz * dz
            w = jnp.maximum(jnp.exp(d * _WSCALE), 0.001)
            den = den + w
            plsc.store_scatter(w_v, [wbase + lanes_k + j], w)
        inv = 1.0 / den
        for i in range(16):
            woff = pl.multiple_of(wbase + i * K, 16)
            w_v[pl.ds(woff, 16)] = w_v[pl.ds(woff, 16)] * inv[i]
            w_v[pl.ds(woff + 16, 16)] = w_v[pl.ds(woff + 16, 16)] * inv[i]
        return carry

    lax.fori_loop(0, SS, _weights_body, 0)

    def _group(g, buf):
        for i in range(G):
            woff = pl.multiple_of((g * G + i) * K, 16)
            wv0 = w_v[pl.ds(woff, 16)]
            wv1 = w_v[pl.ds(woff + 16, 16)]
            acc = [jnp.zeros((16,), jnp.float32) for _ in range(NV)]
            for j in range(K):
                wj = wv0[j] if j < 16 else wv1[j - 16]
                row = i * K + j
                for c in range(NV):
                    acc[c] = acc[c] + wj * rows_v[buf, row, pl.ds(c * 16, 16)]
            for c in range(NV):
                out_v[i, pl.ds(c * 16, 16)] = acc[c]
        pltpu.sync_copy(out_v, out_hbm.at[pl.ds(pbase + g * G, G)])

    def _pool_body(gp, carry):
        g = 2 * gp
        _wait(g, 0, sem0)
        _group(g, 0)

        @pl.when(g + 2 < NGRP)
        def _():
            _start(g + 2, 0, sem0)

        _wait(g + 1, 1, sem1)
        _group(g + 1, 1)

        @pl.when(g + 3 < NGRP)
        def _():
            _start(g + 3, 1, sem1)

        return carry

    lax.fori_loop(0, NGRP // 2, _pool_body, 0)


def kernel(x, x_pad, idx, neighbor_pts, query_pts, W, b, gamma, beta):
    x_padded = jnp.zeros((N_PAD, C), jnp.float32).at[:N].set(x)
    table = _dense(x_padded, W, b.reshape(1, C), gamma.reshape(1, C),
                   beta.reshape(1, C), x_pad)

    pts = jnp.full((N_PAD, 3), 1e6, jnp.float32).at[:N].set(neighbor_pts)
    q = jnp.zeros((N_PAD, 3), jnp.float32).at[:N].set(query_pts)
    idx_pad = jnp.zeros((N_PAD, K), jnp.int32).at[:N].set(idx.astype(jnp.int32))
    idx_flat = idx_pad.reshape(-1)
    idxt_flat = idx_pad.reshape(NW, PW, K).transpose(0, 2, 1).reshape(-1)

    out = _sc_pool(table, idx_flat, idxt_flat,
                   pts[:, 0].copy(), pts[:, 1].copy(), pts[:, 2].copy(),
                   q[:, 0].copy(), q[:, 1].copy(), q[:, 2].copy())
    return out[:N]
